# Initial kernel scaffold; baseline (speedup 1.0000x reference)
import jax
import jax.numpy as jnp
from jax.experimental import pallas as pl


def kernel(noised, mask):
    raise NotImplementedError("write your pallas kernel here")


# speedup vs baseline: 1.1499x; 1.1499x over previous
import functools

import jax
import jax.numpy as jnp
import numpy as np
from jax import lax
from jax.experimental import pallas as pl
from jax.experimental.pallas import tpu as pltpu
from jax.experimental.pallas import tpu_sc as plsc

_C, _H, _W = 3, 512, 512
_rng = np.random.RandomState(0)
_START_X = int(_rng.randint(0, _W + 1)) % _W
_START_Y = int(_rng.randint(0, _H + 1)) % _H
assert _START_X % 16 == 0

_NW = 32
_ROWS = _C * _H
_RPW = _ROWS // _NW
_LANES = 16
_VPR = _W // _LANES

_r = np.arange(_ROWS)
_SRC_ROWS = ((_r // _H) * _H + ((_r % _H) + _START_Y) % _H).astype(np.int32)

_mesh = plsc.VectorSubcoreMesh(core_axis_name="c", subcore_axis_name="s")


@functools.partial(
    pl.kernel,
    out_type=jax.ShapeDtypeStruct((_ROWS, _W), jnp.float32),
    mesh=_mesh,
    scratch_types=[
        pltpu.VMEM((_RPW,), jnp.int32),
        pltpu.VMEM((_RPW, _W), jnp.float32),
        pltpu.VMEM((_RPW, _W), jnp.int32),
        pltpu.VMEM((_RPW, _W), jnp.float32),
        pltpu.SemaphoreType.DMA,
        pltpu.SemaphoreType.DMA,
        pltpu.SemaphoreType.DMA,
    ],
)
def _copy_move_sc(noised_hbm, mask_hbm, idx_hbm, out_hbm,
                  idx_v, noise_v, mask_v, shift_v, sem_n, sem_m, sem_s):
    wid = lax.axis_index("s") * 2 + lax.axis_index("c")
    base = wid * _RPW
    pltpu.sync_copy(idx_hbm.at[pl.ds(base, _RPW)], idx_v)
    cp_n = pltpu.async_copy(noised_hbm.at[pl.ds(base, _RPW)], noise_v, sem_n)
    cp_m = pltpu.async_copy(mask_hbm.at[pl.ds(base, _RPW)], mask_v, sem_m)
    cp_s = pltpu.async_copy(noised_hbm.at[idx_v], shift_v, sem_s)
    cp_n.wait()
    cp_m.wait()
    cp_s.wait()

    def row(i, carry):
        for j in range(_VPR):
            off = (_LANES * j + _START_X) % _W
            m = mask_v[i, pl.ds(_LANES * j, _LANES)]
            s = shift_v[i, pl.ds(off, _LANES)]
            n = noise_v[i, pl.ds(_LANES * j, _LANES)]
            noise_v[i, pl.ds(_LANES * j, _LANES)] = jnp.where(m > 0, s, n)
        return carry

    lax.fori_loop(0, _RPW, row, 0)
    pltpu.sync_copy(noise_v, out_hbm.at[pl.ds(base, _RPW)])


def kernel(noised, mask):
    noised2 = noised.reshape(_ROWS, _W)
    mask2 = mask.reshape(_ROWS, _W)
    out2 = _copy_move_sc(noised2, mask2, jnp.asarray(_SRC_ROWS))
    return out2.reshape(_C, _H, _W)
